# baseline (speedup 1.0000x reference)
import functools

import jax
import jax.numpy as jnp
from jax import lax
from jax.experimental import pallas as pl
from jax.experimental.pallas import tpu as pltpu
from jax.experimental.pallas import tpu_sc as plsc

NUM_EMBEDDINGS = 1000000
EMBEDDING_DIM = 32

NC = 2
NS = 16
NW = NC * NS

NTC = NUM_EMBEDDINGS // 128
LIN_ROWS = NUM_EMBEDDINGS // 4

BATCH = 4096
SEQ = 50
UNITS = SEQ * (BATCH // 128)
UNITS_PER_W = UNITS // NW
IDX_PER_W = UNITS_PER_W * 128


def _iota16():
    return lax.iota(jnp.int32, 16)


def _detile_kernel(wt_hbm, ltail_hbm, lin_hbm, vin, vout, gsems, ssems):
    wid = lax.axis_index("s") * NC + lax.axis_index("c")

    def col_off(rt):
        return pl.multiple_of(rt * 128, 128)

    def row_off(rt):
        return pl.multiple_of(rt * 32, 32)

    def start_reads(rt, b):
        for jt in range(4):
            pltpu.async_copy(
                wt_hbm.at[pl.ds(8 * jt, 8), pl.ds(col_off(rt), 128)],
                vin.at[b, jt],
                gsems.at[b],
            )

    def drain_reads(rt, b):
        for jt in range(4):
            pltpu.make_async_copy(
                wt_hbm.at[pl.ds(8 * jt, 8), pl.ds(col_off(rt), 128)],
                vin.at[b, jt],
                gsems.at[b],
            ).wait()

    def drain_store(rt, b):
        pltpu.make_async_copy(
            vout.at[b],
            lin_hbm.at[pl.ds(row_off(rt), 32)],
            ssems.at[b],
        ).wait()

    jtv = _iota16() >> 3
    jsv = _iota16() & 7

    def transpose(b):
        for c in range(128):
            for h in range(2):
                v = plsc.load_gather(
                    vin.at[b], [jtv + 2 * h, jsv, jnp.full((16,), c, jnp.int32)]
                )
                vout[b, c >> 2, pl.ds((c & 3) * 32 + 16 * h, 16)] = v

    n_units = 244 + jnp.where(wid < 4, 1, 0)

    start_reads(wid, 0)

    def step(k, _):
        rt = wid + 32 * k
        b = k % 2
        nrt = wid + 32 * (k + 1)

        @pl.when(k + 1 < n_units)
        def _():
            start_reads(nrt, 1 - b)

        drain_reads(rt, b)

        @pl.when(k >= 2)
        def _():
            drain_store(wid + 32 * (k - 2), b)

        transpose(b)
        pltpu.async_copy(vout.at[b], lin_hbm.at[pl.ds(row_off(rt), 32)], ssems.at[b])
        return ()

    lax.fori_loop(0, n_units, step, ())

    drain_store(wid + 32 * (n_units - 2), n_units % 2)
    drain_store(wid + 32 * (n_units - 1), (n_units - 1) % 2)

    @pl.when(wid == 31)
    def _():
        pltpu.sync_copy(ltail_hbm, vout.at[0, pl.ds(0, 16)])
        pltpu.sync_copy(vout.at[0, pl.ds(0, 16)], lin_hbm.at[pl.ds(NTC * 32, 16)])


def _gather_kernel(lin_hbm, idx_hbm, out_hbm, idx_v, mv, gv, vout, gsems, osems):
    wid = lax.axis_index("s") * NC + lax.axis_index("c")
    u0 = wid * UNITS_PER_W
    pltpu.sync_copy(idx_hbm.at[pl.ds(u0 * 128, IDX_PER_W)], idx_v)

    def unit_su(i):
        u = u0 + i
        return u >> 5, u & 31

    def rvec(i, blk):
        off = pl.multiple_of(i * 128 + blk * 16, 16)
        return idx_v[pl.ds(off, 16)]

    def compute_mv(i, b):
        for blk in range(8):
            mv[b, pl.ds(blk * 16, 16)] = rvec(i, blk) >> 2

    def start_gather(b):
        pltpu.async_copy(lin_hbm.at[mv.at[b]], gv.at[b], gsems.at[b])

    def drain_gather(b):
        pltpu.make_async_copy(lin_hbm.at[mv.at[b]], gv.at[b], gsems.at[b]).wait()

    def out_copies(i, b, make_only):
        s, bt = unit_su(i)
        boff = pl.multiple_of(bt * 128, 128)
        for jt in range(4):
            src = vout.at[b, jt]
            dst = out_hbm.at[s, pl.ds(8 * jt, 8), pl.ds(boff, 128)]
            if make_only:
                pltpu.make_async_copy(src, dst, osems.at[b]).wait()
            else:
                pltpu.async_copy(src, dst, osems.at[b])

    def transpose_select(i, b):
        for blk in range(8):
            sub = (rvec(i, blk) & 3) * 32
            bsv = jnp.full((16,), blk * 16, jnp.int32) + _iota16()
            for jt in range(4):
                for js in range(8):
                    v = plsc.load_gather(gv.at[b], [bsv, sub + (8 * jt + js)])
                    vout[b, jt, js, pl.ds(blk * 16, 16)] = v

    compute_mv(0, 0)
    start_gather(0)

    def step(k, _):
        for b in range(2):
            i = 2 * k + b

            @pl.when(i + 1 < UNITS_PER_W)
            def _():
                compute_mv(i + 1, 1 - b)
                start_gather(1 - b)

            drain_gather(b)

            @pl.when(i >= 2)
            def _():
                out_copies(i - 2, b, make_only=True)

            transpose_select(i, b)
            out_copies(i, b, make_only=False)
        return ()

    lax.fori_loop(0, UNITS_PER_W // 2, step, ())
    out_copies(UNITS_PER_W - 2, 0, make_only=True)
    out_copies(UNITS_PER_W - 1, 1, make_only=True)


def _mesh():
    return plsc.VectorSubcoreMesh(core_axis_name="c", subcore_axis_name="s")


@jax.jit
def _emb_lookup(x, weight):
    wt = weight.T
    ltail = weight[NTC * 128:].reshape(16, 128)
    idx_flat = x.T.reshape(-1)

    detile = functools.partial(
        pl.kernel,
        mesh=_mesh(),
        out_type=jax.ShapeDtypeStruct((LIN_ROWS, 128), jnp.float32),
        scratch_types=[
            pltpu.VMEM((2, 4, 8, 128), jnp.float32),
            pltpu.VMEM((2, 32, 128), jnp.float32),
            pltpu.SemaphoreType.DMA((2,)),
            pltpu.SemaphoreType.DMA((2,)),
        ],
        compiler_params=pltpu.CompilerParams(
            use_tc_tiling_on_sc=True, needs_layout_passes=False
        ),
    )(_detile_kernel)
    lin2 = detile(wt, ltail)

    gather = functools.partial(
        pl.kernel,
        mesh=_mesh(),
        out_type=jax.ShapeDtypeStruct((SEQ, EMBEDDING_DIM, BATCH), jnp.float32),
        scratch_types=[
            pltpu.VMEM((IDX_PER_W,), jnp.int32),
            pltpu.VMEM((2, 128), jnp.int32),
            pltpu.VMEM((2, 128, 128), jnp.float32),
            pltpu.VMEM((2, 4, 8, 128), jnp.float32),
            pltpu.SemaphoreType.DMA((2,)),
            pltpu.SemaphoreType.DMA((2,)),
        ],
        compiler_params=pltpu.CompilerParams(
            use_tc_tiling_on_sc=True, needs_layout_passes=False
        ),
    )(_gather_kernel)
    outp = gather(lin2, idx_flat)
    return outp.transpose(2, 0, 1)


def kernel(x, weight):
    return _emb_lookup(x, weight)


# speedup vs baseline: 1.1624x; 1.1624x over previous
import functools

import jax
import jax.numpy as jnp
from jax import lax
from jax.experimental import pallas as pl
from jax.experimental.pallas import tpu as pltpu
from jax.experimental.pallas import tpu_sc as plsc

NUM_EMBEDDINGS = 1000000
EMBEDDING_DIM = 32

NC = 2
NS = 16
NW = NC * NS

NTC = NUM_EMBEDDINGS // 128
LIN_WORDS = NUM_EMBEDDINGS * EMBEDDING_DIM

BATCH = 4096
SEQ = 50
UNITS = SEQ * (BATCH // 128)
UNITS_PER_W = UNITS // NW
IDX_PER_W = UNITS_PER_W * 128


def _iota16():
    return lax.iota(jnp.int32, 16)


def _detile_kernel(wt_hbm, ltail_hbm, lin_hbm, vin, vout, gsems, ssems):
    wid = lax.axis_index("s") * NC + lax.axis_index("c")

    def col_off(rt):
        return pl.multiple_of(rt * 128, 128)

    def lin_off(rt):
        return pl.multiple_of(rt * 4096, 4096)

    def start_reads(rt, b):
        for jt in range(4):
            pltpu.async_copy(
                wt_hbm.at[pl.ds(8 * jt, 8), pl.ds(col_off(rt), 128)],
                vin.at[b, jt],
                gsems.at[b],
            )

    def drain_reads(rt, b):
        for jt in range(4):
            pltpu.make_async_copy(
                wt_hbm.at[pl.ds(8 * jt, 8), pl.ds(col_off(rt), 128)],
                vin.at[b, jt],
                gsems.at[b],
            ).wait()

    def drain_store(rt, b):
        pltpu.make_async_copy(
            vout.at[b], lin_hbm.at[pl.ds(lin_off(rt), 4096)], ssems.at[b]
        ).wait()

    scatv = _iota16() * 32

    def transpose(b):
        bv = jnp.full((16,), 0, jnp.int32) + b
        for jt in range(4):
            for js in range(8):
                j = 8 * jt + js
                for k in range(8):
                    v = vin[b, jt, js, pl.ds(16 * k, 16)]
                    plsc.store_scatter(vout, [bv, scatv + (512 * k + j)], v)

    n_units = 244 + jnp.where(wid < 4, 1, 0)

    start_reads(wid, 0)

    def step(k, _):
        rt = wid + 32 * k
        b = k % 2

        @pl.when(k + 1 < n_units)
        def _():
            start_reads(wid + 32 * (k + 1), 1 - b)

        drain_reads(rt, b)

        @pl.when(k >= 2)
        def _():
            drain_store(wid + 32 * (k - 2), b)

        transpose(b)
        pltpu.async_copy(
            vout.at[b], lin_hbm.at[pl.ds(lin_off(rt), 4096)], ssems.at[b]
        )
        return ()

    lax.fori_loop(0, n_units, step, ())

    drain_store(wid + 32 * (n_units - 2), n_units % 2)
    drain_store(wid + 32 * (n_units - 1), (n_units - 1) % 2)

    @pl.when(wid == 31)
    def _():
        pltpu.sync_copy(ltail_hbm, vout.at[0, pl.ds(0, 2048)])
        pltpu.sync_copy(
            vout.at[0, pl.ds(0, 2048)], lin_hbm.at[pl.ds(NTC * 4096, 2048)]
        )


def _gather_kernel(lin_hbm, idx_hbm, out_hbm, idx_v, mv, gv, vout, gsems, osems):
    wid = lax.axis_index("s") * NC + lax.axis_index("c")
    u0 = wid * UNITS_PER_W
    pltpu.sync_copy(idx_hbm.at[pl.ds(u0 * 128, IDX_PER_W)], idx_v)

    def unit_su(i):
        u = u0 + i
        return u >> 5, u & 31

    def compute_mv(i, b):
        off0 = pl.multiple_of(i * 128, 128)
        for blk in range(8):
            mv[b, pl.ds(blk * 16, 16)] = idx_v[pl.ds(off0 + blk * 16, 16)]

    def start_gather(b):
        pltpu.async_copy(lin_hbm.at[mv.at[b]], gv.at[b], gsems.at[b])

    def drain_gather(b):
        pltpu.make_async_copy(lin_hbm.at[mv.at[b]], gv.at[b], gsems.at[b]).wait()

    def out_copies(i, b, make_only):
        s, bt = unit_su(i)
        for jt in range(4):
            src = vout.at[b, pl.ds(jt * 1024, 1024)]
            off = pl.multiple_of((((s * 4) + jt) * 32 + bt) * 1024, 1024)
            dst = out_hbm.at[pl.ds(off, 1024)]
            if make_only:
                pltpu.make_async_copy(src, dst, osems.at[b]).wait()
            else:
                pltpu.async_copy(src, dst, osems.at[b])

    ibase = [_iota16() * 128, (_iota16() + 16) * 128]

    def transpose_select(b):
        bv = jnp.full((16,), b, jnp.int32)
        for bs in range(128):
            for h in range(2):
                v = gv[b, bs, pl.ds(16 * h, 16)]
                plsc.store_scatter(vout, [bv, ibase[h] + bs], v)

    compute_mv(0, 0)
    start_gather(0)

    def step(k, _):
        for b in range(2):
            i = 2 * k + b

            @pl.when(i + 1 < UNITS_PER_W)
            def _():
                compute_mv(i + 1, 1 - b)
                start_gather(1 - b)

            drain_gather(b)

            @pl.when(i >= 2)
            def _():
                out_copies(i - 2, b, make_only=True)

            transpose_select(b)
            out_copies(i, b, make_only=False)
        return ()

    lax.fori_loop(0, UNITS_PER_W // 2, step, ())
    out_copies(UNITS_PER_W - 2, 0, make_only=True)
    out_copies(UNITS_PER_W - 1, 1, make_only=True)


def _mesh():
    return plsc.VectorSubcoreMesh(core_axis_name="c", subcore_axis_name="s")


@jax.jit
def _emb_lookup(x, weight):
    wt = weight.T
    ltail = weight[NTC * 128:].reshape(-1)
    idx_flat = x.T.reshape(-1)

    detile = functools.partial(
        pl.kernel,
        mesh=_mesh(),
        out_type=jax.ShapeDtypeStruct((LIN_WORDS,), jnp.float32),
        scratch_types=[
            pltpu.VMEM((2, 4, 8, 128), jnp.float32),
            pltpu.VMEM((2, 4096), jnp.float32),
            pltpu.SemaphoreType.DMA((2,)),
            pltpu.SemaphoreType.DMA((2,)),
        ],
        compiler_params=pltpu.CompilerParams(
            use_tc_tiling_on_sc=True, needs_layout_passes=False
        ),
    )(_detile_kernel)
    lin1d = detile(wt, ltail)
    lin2d = lin1d.reshape(NUM_EMBEDDINGS, EMBEDDING_DIM)

    gather = functools.partial(
        pl.kernel,
        mesh=_mesh(),
        out_type=jax.ShapeDtypeStruct((SEQ * 4 * (BATCH // 128) * 1024,), jnp.float32),
        scratch_types=[
            pltpu.VMEM((IDX_PER_W,), jnp.int32),
            pltpu.VMEM((2, 128), jnp.int32),
            pltpu.VMEM((2, 128, EMBEDDING_DIM), jnp.float32),
            pltpu.VMEM((2, 4096), jnp.float32),
            pltpu.SemaphoreType.DMA((2,)),
            pltpu.SemaphoreType.DMA((2,)),
        ],
        compiler_params=pltpu.CompilerParams(
            use_tc_tiling_on_sc=False, needs_layout_passes=False
        ),
    )(_gather_kernel)
    out5 = gather(lin2d, idx_flat).reshape(SEQ, 4, BATCH // 128, 8, 128)
    out = out5.transpose(2, 4, 0, 1, 3).reshape(BATCH, SEQ, EMBEDDING_DIM)
    return out


def kernel(x, weight):
    return _emb_lookup(x, weight)


# speedup vs baseline: 1.5463x; 1.3303x over previous
import functools

import jax
import jax.numpy as jnp
from jax import lax
from jax.experimental import pallas as pl
from jax.experimental.pallas import tpu as pltpu
from jax.experimental.pallas import tpu_sc as plsc

NUM_EMBEDDINGS = 1000000
EMBEDDING_DIM = 32

NC = 2
NS = 16
NW = NC * NS

NTC = NUM_EMBEDDINGS // 128
LIN_WORDS = NUM_EMBEDDINGS * EMBEDDING_DIM
LIN_ROWS = NUM_EMBEDDINGS // 4

BATCH = 4096
SEQ = 50
UNITS = SEQ * (BATCH // 128)
UNITS_PER_W = UNITS // NW
IDX_PER_W = UNITS_PER_W * 128


def _iota16():
    return lax.iota(jnp.int32, 16)


def _detile_kernel(wt_hbm, ltail_hbm, lin_hbm, vin, vout, gsems, ssems):
    wid = lax.axis_index("s") * NC + lax.axis_index("c")

    def col_off(rt):
        return pl.multiple_of(rt * 128, 128)

    def lin_off(rt):
        return pl.multiple_of(rt * 4096, 4096)

    def start_reads(rt, b):
        for jt in range(4):
            pltpu.async_copy(
                wt_hbm.at[pl.ds(8 * jt, 8), pl.ds(col_off(rt), 128)],
                vin.at[b, jt],
                gsems.at[b],
            )

    def drain_reads(rt, b):
        for jt in range(4):
            pltpu.make_async_copy(
                wt_hbm.at[pl.ds(8 * jt, 8), pl.ds(col_off(rt), 128)],
                vin.at[b, jt],
                gsems.at[b],
            ).wait()

    def drain_store(rt, b):
        pltpu.make_async_copy(
            vout.at[b], lin_hbm.at[pl.ds(lin_off(rt), 4096)], ssems.at[b]
        ).wait()

    scatv = _iota16() * 32

    def transpose(b):
        bv = jnp.full((16,), 0, jnp.int32) + b
        for jt in range(4):
            for js in range(8):
                j = 8 * jt + js
                for k in range(8):
                    v = vin[b, jt, js, pl.ds(16 * k, 16)]
                    plsc.store_scatter(vout, [bv, scatv + (512 * k + j)], v)

    n_units = 244 + jnp.where(wid < 4, 1, 0)

    start_reads(wid, 0)

    def step(k, _):
        rt = wid + 32 * k
        b = k % 2

        @pl.when(k + 1 < n_units)
        def _():
            start_reads(wid + 32 * (k + 1), 1 - b)

        drain_reads(rt, b)

        @pl.when(k >= 2)
        def _():
            drain_store(wid + 32 * (k - 2), b)

        transpose(b)
        pltpu.async_copy(
            vout.at[b], lin_hbm.at[pl.ds(lin_off(rt), 4096)], ssems.at[b]
        )
        return ()

    lax.fori_loop(0, n_units, step, ())

    drain_store(wid + 32 * (n_units - 2), n_units % 2)
    drain_store(wid + 32 * (n_units - 1), (n_units - 1) % 2)

    @pl.when(wid == 31)
    def _():
        pltpu.sync_copy(ltail_hbm, vout.at[0, pl.ds(0, 2048)])
        pltpu.sync_copy(
            vout.at[0, pl.ds(0, 2048)], lin_hbm.at[pl.ds(NTC * 4096, 2048)]
        )


def _gather_kernel(lin_hbm, idx_hbm, out_hbm, idx_v, mv, gv, vout, gsems, osems):
    wid = lax.axis_index("s") * NC + lax.axis_index("c")
    u0 = wid * UNITS_PER_W
    pltpu.sync_copy(idx_hbm.at[pl.ds(u0 * 128, IDX_PER_W)], idx_v)

    def unit_su(i):
        u = u0 + i
        return u >> 5, u & 31

    def compute_mv(i, b):
        off0 = pl.multiple_of(i * 128, 128)
        for blk in range(8):
            mv[b, pl.ds(blk * 16, 16)] = idx_v[pl.ds(off0 + blk * 16, 16)]

    def start_gather(b):
        pltpu.async_copy(lin_hbm.at[mv.at[b]], gv.at[b], gsems.at[b])

    def drain_gather(b):
        pltpu.make_async_copy(lin_hbm.at[mv.at[b]], gv.at[b], gsems.at[b]).wait()

    def out_copies(i, b, make_only):
        s, bt = unit_su(i)
        for jt in range(4):
            src = vout.at[b, pl.ds(jt * 1024, 1024)]
            off = pl.multiple_of((((s * 4) + jt) * 32 + bt) * 1024, 1024)
            dst = out_hbm.at[pl.ds(off, 1024)]
            if make_only:
                pltpu.make_async_copy(src, dst, osems.at[b]).wait()
            else:
                pltpu.async_copy(src, dst, osems.at[b])

    ibase = [_iota16() * 128, (_iota16() + 16) * 128]

    def transpose_select(b):
        bv = jnp.full((16,), b, jnp.int32)
        for bs in range(128):
            for h in range(2):
                v = gv[b, bs, pl.ds(16 * h, 16)]
                plsc.store_scatter(vout, [bv, ibase[h] + bs], v)

    compute_mv(0, 0)
    start_gather(0)

    def step(k, _):
        for b in range(2):
            i = 2 * k + b

            @pl.when(i + 1 < UNITS_PER_W)
            def _():
                compute_mv(i + 1, 1 - b)
                start_gather(1 - b)

            drain_gather(b)

            @pl.when(i >= 2)
            def _():
                out_copies(i - 2, b, make_only=True)

            transpose_select(b)
            out_copies(i, b, make_only=False)
        return ()

    lax.fori_loop(0, UNITS_PER_W // 2, step, ())
    out_copies(UNITS_PER_W - 2, 0, make_only=True)
    out_copies(UNITS_PER_W - 1, 1, make_only=True)


def _mesh():
    return plsc.VectorSubcoreMesh(core_axis_name="c", subcore_axis_name="s")


TCC = 2048


def _tc_detile_kernel(wt_ref, lin_ref):
    x = wt_ref[...]
    z = x.T.reshape(TCC // 4, 4, 32)
    parts = [z[:, q, :] for q in range(4)]
    lin_ref[...] = jnp.concatenate(parts, axis=1)


def _tc_detile(wt):
    grid = (NUM_EMBEDDINGS + TCC - 1) // TCC
    return pl.pallas_call(
        _tc_detile_kernel,
        grid=(grid,),
        in_specs=[pl.BlockSpec((32, TCC), lambda i: (0, i))],
        out_specs=pl.BlockSpec((TCC // 4, 128), lambda i: (i, 0)),
        out_shape=jax.ShapeDtypeStruct((LIN_ROWS, 128), jnp.float32),
    )(wt)


@jax.jit
def _emb_lookup(x, weight):
    wt = weight.T
    ltail = weight[NTC * 128:].reshape(-1)
    idx_flat = x.T.reshape(-1)

    lin2d = _tc_detile(wt).reshape(NUM_EMBEDDINGS, EMBEDDING_DIM)

    gather = functools.partial(
        pl.kernel,
        mesh=_mesh(),
        out_type=jax.ShapeDtypeStruct((SEQ * 4 * (BATCH // 128) * 1024,), jnp.float32),
        scratch_types=[
            pltpu.VMEM((IDX_PER_W,), jnp.int32),
            pltpu.VMEM((2, 128), jnp.int32),
            pltpu.VMEM((2, 128, EMBEDDING_DIM), jnp.float32),
            pltpu.VMEM((2, 4096), jnp.float32),
            pltpu.SemaphoreType.DMA((2,)),
            pltpu.SemaphoreType.DMA((2,)),
        ],
        compiler_params=pltpu.CompilerParams(
            use_tc_tiling_on_sc=False, needs_layout_passes=False
        ),
    )(_gather_kernel)
    out5 = gather(lin2d, idx_flat).reshape(SEQ, 4, BATCH // 128, 8, 128)
    out = out5.transpose(2, 4, 0, 1, 3).reshape(BATCH, SEQ, EMBEDDING_DIM)
    return out


def kernel(x, weight):
    return _emb_lookup(x, weight)


# speedup vs baseline: 1.7205x; 1.1126x over previous
import functools

import jax
import jax.numpy as jnp
from jax import lax
from jax.experimental import pallas as pl
from jax.experimental.pallas import tpu as pltpu
from jax.experimental.pallas import tpu_sc as plsc

NUM_EMBEDDINGS = 1000000
EMBEDDING_DIM = 32

NC = 2
NS = 16
NW = NC * NS

NTC = NUM_EMBEDDINGS // 128
LIN_WORDS = NUM_EMBEDDINGS * EMBEDDING_DIM
LIN_ROWS = NUM_EMBEDDINGS // 4

BATCH = 4096
SEQ = 50
UNITS = SEQ * (BATCH // 128)
UNITS_PER_W = UNITS // NW
IDX_PER_W = UNITS_PER_W * 128


def _iota16():
    return lax.iota(jnp.int32, 16)


def _detile_kernel(wt_hbm, ltail_hbm, lin_hbm, vin, vout, gsems, ssems):
    wid = lax.axis_index("s") * NC + lax.axis_index("c")

    def col_off(rt):
        return pl.multiple_of(rt * 128, 128)

    def lin_off(rt):
        return pl.multiple_of(rt * 4096, 4096)

    def start_reads(rt, b):
        for jt in range(4):
            pltpu.async_copy(
                wt_hbm.at[pl.ds(8 * jt, 8), pl.ds(col_off(rt), 128)],
                vin.at[b, jt],
                gsems.at[b],
            )

    def drain_reads(rt, b):
        for jt in range(4):
            pltpu.make_async_copy(
                wt_hbm.at[pl.ds(8 * jt, 8), pl.ds(col_off(rt), 128)],
                vin.at[b, jt],
                gsems.at[b],
            ).wait()

    def drain_store(rt, b):
        pltpu.make_async_copy(
            vout.at[b], lin_hbm.at[pl.ds(lin_off(rt), 4096)], ssems.at[b]
        ).wait()

    scatv = _iota16() * 32

    def transpose(b):
        bv = jnp.full((16,), 0, jnp.int32) + b
        for jt in range(4):
            for js in range(8):
                j = 8 * jt + js
                for k in range(8):
                    v = vin[b, jt, js, pl.ds(16 * k, 16)]
                    plsc.store_scatter(vout, [bv, scatv + (512 * k + j)], v)

    n_units = 244 + jnp.where(wid < 4, 1, 0)

    start_reads(wid, 0)

    def step(k, _):
        rt = wid + 32 * k
        b = k % 2

        @pl.when(k + 1 < n_units)
        def _():
            start_reads(wid + 32 * (k + 1), 1 - b)

        drain_reads(rt, b)

        @pl.when(k >= 2)
        def _():
            drain_store(wid + 32 * (k - 2), b)

        transpose(b)
        pltpu.async_copy(
            vout.at[b], lin_hbm.at[pl.ds(lin_off(rt), 4096)], ssems.at[b]
        )
        return ()

    lax.fori_loop(0, n_units, step, ())

    drain_store(wid + 32 * (n_units - 2), n_units % 2)
    drain_store(wid + 32 * (n_units - 1), (n_units - 1) % 2)

    @pl.when(wid == 31)
    def _():
        pltpu.sync_copy(ltail_hbm, vout.at[0, pl.ds(0, 2048)])
        pltpu.sync_copy(
            vout.at[0, pl.ds(0, 2048)], lin_hbm.at[pl.ds(NTC * 4096, 2048)]
        )


def _gather_kernel(lin_hbm, idx_hbm, out_hbm, idx_v, mv, gv, vout, gsems, osems):
    wid = lax.axis_index("s") * NC + lax.axis_index("c")
    u0 = wid * UNITS_PER_W
    pltpu.sync_copy(idx_hbm.at[pl.ds(u0 * 128, IDX_PER_W)], idx_v)

    def unit_su(i):
        u = u0 + i
        return u >> 5, u & 31

    def compute_mv(i, b):
        off0 = pl.multiple_of(i * 128, 128)
        for blk in range(8):
            r = idx_v[pl.ds(off0 + blk * 16, 16)]
            c2 = r & 2047
            rho = (r - c2) + ((c2 & 511) << 2) + (c2 >> 9)
            mv[b, pl.ds(blk * 16, 16)] = rho

    def start_gather(b):
        pltpu.async_copy(lin_hbm.at[mv.at[b]], gv.at[b], gsems.at[b])

    def drain_gather(b):
        pltpu.make_async_copy(lin_hbm.at[mv.at[b]], gv.at[b], gsems.at[b]).wait()

    def out_copies(i, b, make_only):
        s, bt = unit_su(i)
        for jt in range(4):
            src = vout.at[b, pl.ds(jt * 1024, 1024)]
            off = pl.multiple_of((((s * 4) + jt) * 32 + bt) * 1024, 1024)
            dst = out_hbm.at[pl.ds(off, 1024)]
            if make_only:
                pltpu.make_async_copy(src, dst, osems.at[b]).wait()
            else:
                pltpu.async_copy(src, dst, osems.at[b])

    ibase = [_iota16() * 128, (_iota16() + 16) * 128]

    def transpose_select(b):
        bv = jnp.full((16,), b, jnp.int32)
        for bs in range(128):
            for h in range(2):
                v = gv[b, bs, pl.ds(16 * h, 16)]
                plsc.store_scatter(vout, [bv, ibase[h] + bs], v)

    compute_mv(0, 0)
    start_gather(0)

    def step(k, _):
        for b in range(2):
            i = 2 * k + b

            @pl.when(i + 1 < UNITS_PER_W)
            def _():
                compute_mv(i + 1, 1 - b)
                start_gather(1 - b)

            drain_gather(b)

            @pl.when(i >= 2)
            def _():
                out_copies(i - 2, b, make_only=True)

            transpose_select(b)
            out_copies(i, b, make_only=False)
        return ()

    lax.fori_loop(0, UNITS_PER_W // 2, step, ())
    out_copies(UNITS_PER_W - 2, 0, make_only=True)
    out_copies(UNITS_PER_W - 1, 1, make_only=True)


def _mesh():
    return plsc.VectorSubcoreMesh(core_axis_name="c", subcore_axis_name="s")


TCC = 2048
TCG = (NUM_EMBEDDINGS + TCC - 1) // TCC
LIN_PAD_ROWS = TCG * (TCC // 4)


def _tc_detile_kernel(wt_ref, lin_ref):
    x = wt_ref[...]
    parts = [x[:, g * 512:(g + 1) * 512].T for g in range(4)]
    lin_ref[...] = jnp.concatenate(parts, axis=1)


def _tc_detile(wt):
    return pl.pallas_call(
        _tc_detile_kernel,
        grid=(TCG,),
        in_specs=[pl.BlockSpec((32, TCC), lambda i: (0, i))],
        out_specs=pl.BlockSpec((TCC // 4, 128), lambda i: (i, 0)),
        out_shape=jax.ShapeDtypeStruct((LIN_PAD_ROWS, 128), jnp.float32),
    )(wt)


@jax.jit
def _emb_lookup(x, weight):
    wt = weight.T
    ltail = weight[NTC * 128:].reshape(-1)
    idx_flat = x.T.reshape(-1)

    lin2d = _tc_detile(wt).reshape(LIN_PAD_ROWS * 4, EMBEDDING_DIM)

    gather = functools.partial(
        pl.kernel,
        mesh=_mesh(),
        out_type=jax.ShapeDtypeStruct((SEQ * 4 * (BATCH // 128) * 1024,), jnp.float32),
        scratch_types=[
            pltpu.VMEM((IDX_PER_W,), jnp.int32),
            pltpu.VMEM((2, 128), jnp.int32),
            pltpu.VMEM((2, 128, EMBEDDING_DIM), jnp.float32),
            pltpu.VMEM((2, 4096), jnp.float32),
            pltpu.SemaphoreType.DMA((2,)),
            pltpu.SemaphoreType.DMA((2,)),
        ],
        compiler_params=pltpu.CompilerParams(
            use_tc_tiling_on_sc=False, needs_layout_passes=False
        ),
    )(_gather_kernel)
    out5 = gather(lin2d, idx_flat).reshape(SEQ, 4, BATCH // 128, 8, 128)
    out = out5.transpose(2, 4, 0, 1, 3).reshape(BATCH, SEQ, EMBEDDING_DIM)
    return out


def kernel(x, weight):
    return _emb_lookup(x, weight)


# speedup vs baseline: 2.4224x; 1.4079x over previous
import functools

import jax
import jax.numpy as jnp
from jax import lax
from jax.experimental import pallas as pl
from jax.experimental.pallas import tpu as pltpu
from jax.experimental.pallas import tpu_sc as plsc

NUM_EMBEDDINGS = 1000000
EMBEDDING_DIM = 32

NC = 2
NS = 16
NW = NC * NS

NTC = NUM_EMBEDDINGS // 128
LIN_WORDS = NUM_EMBEDDINGS * EMBEDDING_DIM
LIN_ROWS = NUM_EMBEDDINGS // 4

BATCH = 4096
SEQ = 50
UNITS = SEQ * (BATCH // 128)
UNITS_PER_W = UNITS // NW
TCC = 8192
TCQ_SHIFT = 11
IDX_PER_W = UNITS_PER_W * 128


def _iota16():
    return lax.iota(jnp.int32, 16)


def _detile_kernel(wt_hbm, ltail_hbm, lin_hbm, vin, vout, gsems, ssems):
    wid = lax.axis_index("s") * NC + lax.axis_index("c")

    def col_off(rt):
        return pl.multiple_of(rt * 128, 128)

    def lin_off(rt):
        return pl.multiple_of(rt * 4096, 4096)

    def start_reads(rt, b):
        for jt in range(4):
            pltpu.async_copy(
                wt_hbm.at[pl.ds(8 * jt, 8), pl.ds(col_off(rt), 128)],
                vin.at[b, jt],
                gsems.at[b],
            )

    def drain_reads(rt, b):
        for jt in range(4):
            pltpu.make_async_copy(
                wt_hbm.at[pl.ds(8 * jt, 8), pl.ds(col_off(rt), 128)],
                vin.at[b, jt],
                gsems.at[b],
            ).wait()

    def drain_store(rt, b):
        pltpu.make_async_copy(
            vout.at[b], lin_hbm.at[pl.ds(lin_off(rt), 4096)], ssems.at[b]
        ).wait()

    scatv = _iota16() * 32

    def transpose(b):
        bv = jnp.full((16,), 0, jnp.int32) + b
        for jt in range(4):
            for js in range(8):
                j = 8 * jt + js
                for k in range(8):
                    v = vin[b, jt, js, pl.ds(16 * k, 16)]
                    plsc.store_scatter(vout, [bv, scatv + (512 * k + j)], v)

    n_units = 244 + jnp.where(wid < 4, 1, 0)

    start_reads(wid, 0)

    def step(k, _):
        rt = wid + 32 * k
        b = k % 2

        @pl.when(k + 1 < n_units)
        def _():
            start_reads(wid + 32 * (k + 1), 1 - b)

        drain_reads(rt, b)

        @pl.when(k >= 2)
        def _():
            drain_store(wid + 32 * (k - 2), b)

        transpose(b)
        pltpu.async_copy(
            vout.at[b], lin_hbm.at[pl.ds(lin_off(rt), 4096)], ssems.at[b]
        )
        return ()

    lax.fori_loop(0, n_units, step, ())

    drain_store(wid + 32 * (n_units - 2), n_units % 2)
    drain_store(wid + 32 * (n_units - 1), (n_units - 1) % 2)

    @pl.when(wid == 31)
    def _():
        pltpu.sync_copy(ltail_hbm, vout.at[0, pl.ds(0, 2048)])
        pltpu.sync_copy(
            vout.at[0, pl.ds(0, 2048)], lin_hbm.at[pl.ds(NTC * 4096, 2048)]
        )


def _gather_kernel(lin_hbm, idx_hbm, out_hbm, idx_v, mv, gv, vout, gsems, osems):
    wid = lax.axis_index("s") * NC + lax.axis_index("c")
    u0 = wid * UNITS_PER_W
    pltpu.sync_copy(idx_hbm.at[pl.ds(u0 * 128, IDX_PER_W)], idx_v)

    def unit_su(i):
        u = u0 + i
        return u >> 5, u & 31

    def compute_mv(i, b):
        off0 = pl.multiple_of(i * 128, 128)
        for blk in range(8):
            r = idx_v[pl.ds(off0 + blk * 16, 16)]
            c2 = r & (TCC - 1)
            rho = (r - c2) + ((c2 & (TCC // 4 - 1)) << 2) + (c2 >> TCQ_SHIFT)
            mv[b, pl.ds(blk * 16, 16)] = rho

    def start_gather(b):
        pltpu.async_copy(lin_hbm.at[mv.at[b]], gv.at[b], gsems.at[b])

    def drain_gather(b):
        pltpu.make_async_copy(lin_hbm.at[mv.at[b]], gv.at[b], gsems.at[b]).wait()

    def out_copies(i, b, make_only):
        s, bt = unit_su(i)
        for jt in range(4):
            src = vout.at[b, pl.ds(jt * 1024, 1024)]
            off = pl.multiple_of((((s * 4) + jt) * 32 + bt) * 1024, 1024)
            dst = out_hbm.at[pl.ds(off, 1024)]
            if make_only:
                pltpu.make_async_copy(src, dst, osems.at[b]).wait()
            else:
                pltpu.async_copy(src, dst, osems.at[b])

    ibase = [_iota16() * 128, (_iota16() + 16) * 128]

    def transpose_select(b):
        bv = jnp.full((16,), b, jnp.int32)
        for bs in range(128):
            for h in range(2):
                v = gv[b, bs, pl.ds(16 * h, 16)]
                plsc.store_scatter(vout, [bv, ibase[h] + bs], v)

    compute_mv(0, 0)
    start_gather(0)

    def step(k, _):
        for b in range(2):
            i = 2 * k + b

            @pl.when(i + 1 < UNITS_PER_W)
            def _():
                compute_mv(i + 1, 1 - b)
                start_gather(1 - b)

            drain_gather(b)

            @pl.when(i >= 2)
            def _():
                out_copies(i - 2, b, make_only=True)

            transpose_select(b)
            out_copies(i, b, make_only=False)
        return ()

    lax.fori_loop(0, UNITS_PER_W // 2, step, ())
    out_copies(UNITS_PER_W - 2, 0, make_only=True)
    out_copies(UNITS_PER_W - 1, 1, make_only=True)


def _mesh():
    return plsc.VectorSubcoreMesh(core_axis_name="c", subcore_axis_name="s")


TCG = (NUM_EMBEDDINGS + TCC - 1) // TCC
LIN_PAD_ROWS = TCG * (TCC // 4)


def _tc_detile_kernel(wt_ref, lin_ref):
    x = wt_ref[...]
    parts = [x[:, g * (TCC // 4):(g + 1) * (TCC // 4)].T for g in range(4)]
    lin_ref[...] = jnp.concatenate(parts, axis=1)


def _tc_detile(wt):
    return pl.pallas_call(
        _tc_detile_kernel,
        grid=(TCG,),
        in_specs=[pl.BlockSpec((32, TCC), lambda i: (0, i))],
        out_specs=pl.BlockSpec((TCC // 4, 128), lambda i: (i, 0)),
        out_shape=jax.ShapeDtypeStruct((LIN_PAD_ROWS, 128), jnp.float32),
    )(wt)


@jax.jit
def _emb_lookup(x, weight):
    wt = weight.T
    ltail = weight[NTC * 128:].reshape(-1)
    idx_flat = x.T.reshape(-1)

    lin2d = _tc_detile(wt).reshape(LIN_PAD_ROWS * 4, EMBEDDING_DIM)

    gather = functools.partial(
        pl.kernel,
        mesh=_mesh(),
        out_type=jax.ShapeDtypeStruct((SEQ * 4 * (BATCH // 128) * 1024,), jnp.float32),
        scratch_types=[
            pltpu.VMEM((IDX_PER_W,), jnp.int32),
            pltpu.VMEM((2, 128), jnp.int32),
            pltpu.VMEM((2, 128, EMBEDDING_DIM), jnp.float32),
            pltpu.VMEM((2, 4096), jnp.float32),
            pltpu.SemaphoreType.DMA((2,)),
            pltpu.SemaphoreType.DMA((2,)),
        ],
        compiler_params=pltpu.CompilerParams(
            use_tc_tiling_on_sc=False, needs_layout_passes=False
        ),
    )(_gather_kernel)
    out5 = gather(lin2d, idx_flat).reshape(SEQ, 4, BATCH // 128, 8, 128)
    out = out5.transpose(2, 4, 0, 1, 3).reshape(BATCH, SEQ, EMBEDDING_DIM)
    return out


def kernel(x, weight):
    return _emb_lookup(x, weight)


# speedup vs baseline: 2.4608x; 1.0159x over previous
import functools

import jax
import jax.numpy as jnp
from jax import lax
from jax.experimental import pallas as pl
from jax.experimental.pallas import tpu as pltpu
from jax.experimental.pallas import tpu_sc as plsc

NUM_EMBEDDINGS = 1000000
EMBEDDING_DIM = 32

NC = 2
NS = 16
NW = NC * NS

NTC = NUM_EMBEDDINGS // 128
LIN_WORDS = NUM_EMBEDDINGS * EMBEDDING_DIM
LIN_ROWS = NUM_EMBEDDINGS // 4

BATCH = 4096
SEQ = 50
UNITS = SEQ * (BATCH // 128)
UNITS_PER_W = UNITS // NW
TCC = 32768
TCQ_SHIFT = 13
IDX_PER_W = UNITS_PER_W * 128


def _iota16():
    return lax.iota(jnp.int32, 16)


def _detile_kernel(wt_hbm, ltail_hbm, lin_hbm, vin, vout, gsems, ssems):
    wid = lax.axis_index("s") * NC + lax.axis_index("c")

    def col_off(rt):
        return pl.multiple_of(rt * 128, 128)

    def lin_off(rt):
        return pl.multiple_of(rt * 4096, 4096)

    def start_reads(rt, b):
        for jt in range(4):
            pltpu.async_copy(
                wt_hbm.at[pl.ds(8 * jt, 8), pl.ds(col_off(rt), 128)],
                vin.at[b, jt],
                gsems.at[b],
            )

    def drain_reads(rt, b):
        for jt in range(4):
            pltpu.make_async_copy(
                wt_hbm.at[pl.ds(8 * jt, 8), pl.ds(col_off(rt), 128)],
                vin.at[b, jt],
                gsems.at[b],
            ).wait()

    def drain_store(rt, b):
        pltpu.make_async_copy(
            vout.at[b], lin_hbm.at[pl.ds(lin_off(rt), 4096)], ssems.at[b]
        ).wait()

    scatv = _iota16() * 32

    def transpose(b):
        bv = jnp.full((16,), 0, jnp.int32) + b
        for jt in range(4):
            for js in range(8):
                j = 8 * jt + js
                for k in range(8):
                    v = vin[b, jt, js, pl.ds(16 * k, 16)]
                    plsc.store_scatter(vout, [bv, scatv + (512 * k + j)], v)

    n_units = 244 + jnp.where(wid < 4, 1, 0)

    start_reads(wid, 0)

    def step(k, _):
        rt = wid + 32 * k
        b = k % 2

        @pl.when(k + 1 < n_units)
        def _():
            start_reads(wid + 32 * (k + 1), 1 - b)

        drain_reads(rt, b)

        @pl.when(k >= 2)
        def _():
            drain_store(wid + 32 * (k - 2), b)

        transpose(b)
        pltpu.async_copy(
            vout.at[b], lin_hbm.at[pl.ds(lin_off(rt), 4096)], ssems.at[b]
        )
        return ()

    lax.fori_loop(0, n_units, step, ())

    drain_store(wid + 32 * (n_units - 2), n_units % 2)
    drain_store(wid + 32 * (n_units - 1), (n_units - 1) % 2)

    @pl.when(wid == 31)
    def _():
        pltpu.sync_copy(ltail_hbm, vout.at[0, pl.ds(0, 2048)])
        pltpu.sync_copy(
            vout.at[0, pl.ds(0, 2048)], lin_hbm.at[pl.ds(NTC * 4096, 2048)]
        )


def _gather_kernel(lin_hbm, idx_hbm, out_hbm, idx_v, mv, gv, vout, gsems, osems):
    wid = lax.axis_index("s") * NC + lax.axis_index("c")
    u0 = wid * UNITS_PER_W
    pltpu.sync_copy(idx_hbm.at[pl.ds(u0 * 128, IDX_PER_W)], idx_v)

    def unit_su(i):
        u = u0 + i
        return u >> 5, u & 31

    def compute_mv(i, b):
        off0 = pl.multiple_of(i * 128, 128)
        for blk in range(8):
            r = idx_v[pl.ds(off0 + blk * 16, 16)]
            c2 = r & (TCC - 1)
            rho = (r - c2) + ((c2 & (TCC // 4 - 1)) << 2) + (c2 >> TCQ_SHIFT)
            mv[b, pl.ds(blk * 16, 16)] = rho

    def start_gather(b):
        pltpu.async_copy(lin_hbm.at[mv.at[b]], gv.at[b], gsems.at[b])

    def drain_gather(b):
        pltpu.make_async_copy(lin_hbm.at[mv.at[b]], gv.at[b], gsems.at[b]).wait()

    def out_copies(i, b, make_only):
        s, bt = unit_su(i)
        for jt in range(4):
            src = vout.at[b, pl.ds(jt * 1024, 1024)]
            off = pl.multiple_of((((s * 4) + jt) * 32 + bt) * 1024, 1024)
            dst = out_hbm.at[pl.ds(off, 1024)]
            if make_only:
                pltpu.make_async_copy(src, dst, osems.at[b]).wait()
            else:
                pltpu.async_copy(src, dst, osems.at[b])

    ibase = [_iota16() * 128, (_iota16() + 16) * 128]

    def transpose_select(b):
        bv = jnp.full((16,), b, jnp.int32)
        for bs in range(128):
            for h in range(2):
                v = gv[b, bs, pl.ds(16 * h, 16)]
                plsc.store_scatter(vout, [bv, ibase[h] + bs], v)

    compute_mv(0, 0)
    start_gather(0)

    def step(k, _):
        for b in range(2):
            i = 2 * k + b

            @pl.when(i + 1 < UNITS_PER_W)
            def _():
                compute_mv(i + 1, 1 - b)
                start_gather(1 - b)

            drain_gather(b)

            @pl.when(i >= 2)
            def _():
                out_copies(i - 2, b, make_only=True)

            transpose_select(b)
            out_copies(i, b, make_only=False)
        return ()

    lax.fori_loop(0, UNITS_PER_W // 2, step, ())
    out_copies(UNITS_PER_W - 2, 0, make_only=True)
    out_copies(UNITS_PER_W - 1, 1, make_only=True)


def _mesh():
    return plsc.VectorSubcoreMesh(core_axis_name="c", subcore_axis_name="s")


TCG = (NUM_EMBEDDINGS + TCC - 1) // TCC
LIN_PAD_ROWS = TCG * (TCC // 4)


def _tc_detile_kernel(wt_ref, lin_ref):
    x = wt_ref[...]
    parts = [x[:, g * (TCC // 4):(g + 1) * (TCC // 4)].T for g in range(4)]
    lin_ref[...] = jnp.concatenate(parts, axis=1)


def _tc_detile(wt):
    return pl.pallas_call(
        _tc_detile_kernel,
        grid=(TCG,),
        in_specs=[pl.BlockSpec((32, TCC), lambda i: (0, i))],
        out_specs=pl.BlockSpec((TCC // 4, 128), lambda i: (i, 0)),
        out_shape=jax.ShapeDtypeStruct((LIN_PAD_ROWS, 128), jnp.float32),
    )(wt)


@jax.jit
def _emb_lookup(x, weight):
    wt = weight.T
    ltail = weight[NTC * 128:].reshape(-1)
    idx_flat = x.T.reshape(-1)

    lin2d = _tc_detile(wt).reshape(LIN_PAD_ROWS * 4, EMBEDDING_DIM)

    gather = functools.partial(
        pl.kernel,
        mesh=_mesh(),
        out_type=jax.ShapeDtypeStruct((SEQ * 4 * (BATCH // 128) * 1024,), jnp.float32),
        scratch_types=[
            pltpu.VMEM((IDX_PER_W,), jnp.int32),
            pltpu.VMEM((2, 128), jnp.int32),
            pltpu.VMEM((2, 128, EMBEDDING_DIM), jnp.float32),
            pltpu.VMEM((2, 4096), jnp.float32),
            pltpu.SemaphoreType.DMA((2,)),
            pltpu.SemaphoreType.DMA((2,)),
        ],
        compiler_params=pltpu.CompilerParams(
            use_tc_tiling_on_sc=False, needs_layout_passes=False
        ),
    )(_gather_kernel)
    out5 = gather(lin2d, idx_flat).reshape(SEQ, 4, BATCH // 128, 8, 128)
    out = out5.transpose(2, 4, 0, 1, 3).reshape(BATCH, SEQ, EMBEDDING_DIM)
    return out


def kernel(x, weight):
    return _emb_lookup(x, weight)
